# Initial kernel scaffold; baseline (speedup 1.0000x reference)
import jax
import jax.numpy as jnp
from jax.experimental import pallas as pl


def kernel(src, dest, edge_attr, u, batch, W1, b1, W2, b2):
    raise NotImplementedError("write your pallas kernel here")


# speedup vs baseline: 2.1423x; 2.1423x over previous
import jax
import jax.numpy as jnp
from jax.experimental import pallas as pl
from jax.experimental.pallas import tpu as pltpu

NODE_F = 128
EDGE_F = 16
HIDDEN = 128
BE = 2560


def _edge_mlp_kernel(batch_ref, src_ref, dest_ref, ea_ref, up_ref,
                     w1a_ref, w1b_ref, w1c_ref, w2_ref, b2_ref, out_ref):
    g = up_ref.shape[0]
    idx = batch_ref[0, 0, :].reshape(BE, 1)
    onehot = (idx == jax.lax.broadcasted_iota(jnp.int32, (BE, g), 1)
              ).astype(jnp.float32)
    acc = jnp.dot(src_ref[...], w1a_ref[...], preferred_element_type=jnp.float32)
    acc += jnp.dot(dest_ref[...], w1b_ref[...], preferred_element_type=jnp.float32)
    acc += jnp.dot(ea_ref[...], w1c_ref[...], preferred_element_type=jnp.float32)
    acc += jnp.dot(onehot, up_ref[...], preferred_element_type=jnp.float32)
    h = jnp.maximum(acc, 0.0)
    out_ref[...] = jnp.dot(h, w2_ref[...], preferred_element_type=jnp.float32) + b2_ref[...]


def kernel(src, dest, edge_attr, u, batch, W1, b1, W2, b2):
    e = src.shape[0]
    g = u.shape[0]
    nb = e // BE
    w1a = W1[:NODE_F]
    w1b = W1[NODE_F:2 * NODE_F]
    w1c = W1[2 * NODE_F:2 * NODE_F + EDGE_F]
    w1d = W1[2 * NODE_F + EDGE_F:]
    up = u @ w1d + b1[None, :]
    batch3 = batch.astype(jnp.int32).reshape(nb, 1, BE)
    b2r = b2.reshape(1, EDGE_F)

    grid_spec = pl.GridSpec(
        grid=(nb,),
        in_specs=[
            pl.BlockSpec((1, 1, BE), lambda i: (i, 0, 0)),
            pl.BlockSpec((BE, NODE_F), lambda i: (i, 0)),
            pl.BlockSpec((BE, NODE_F), lambda i: (i, 0)),
            pl.BlockSpec((BE, EDGE_F), lambda i: (i, 0)),
            pl.BlockSpec((g, HIDDEN), lambda i: (0, 0)),
            pl.BlockSpec((NODE_F, HIDDEN), lambda i: (0, 0)),
            pl.BlockSpec((NODE_F, HIDDEN), lambda i: (0, 0)),
            pl.BlockSpec((EDGE_F, HIDDEN), lambda i: (0, 0)),
            pl.BlockSpec((HIDDEN, EDGE_F), lambda i: (0, 0)),
            pl.BlockSpec((1, EDGE_F), lambda i: (0, 0)),
        ],
        out_specs=pl.BlockSpec((BE, EDGE_F), lambda i: (i, 0)),
    )
    return pl.pallas_call(
        _edge_mlp_kernel,
        grid_spec=grid_spec,
        out_shape=jax.ShapeDtypeStruct((e, EDGE_F), jnp.float32),
        compiler_params=pltpu.CompilerParams(
            dimension_semantics=("arbitrary",),
        ),
    )(batch3, src, dest, edge_attr, up, w1a, w1b, w1c, W2, b2r)


# speedup vs baseline: 2.2517x; 1.0511x over previous
import jax
import jax.numpy as jnp
from jax.experimental import pallas as pl
from jax.experimental.pallas import tpu as pltpu

NODE_F = 128
EDGE_F = 16
HIDDEN = 128
BE = 5000


def _edge_mlp_kernel(batch_ref, src_ref, dest_ref, ea_ref, up_ref,
                     w1a_ref, w1b_ref, w1c_ref, w2_ref, b2_ref, out_ref):
    g = up_ref.shape[0]
    idx = batch_ref[0, 0, :].reshape(BE, 1)
    onehot = (idx == jax.lax.broadcasted_iota(jnp.int32, (BE, g), 1)
              ).astype(jnp.float32)
    acc = jnp.dot(src_ref[...], w1a_ref[...], preferred_element_type=jnp.float32)
    acc += jnp.dot(dest_ref[...], w1b_ref[...], preferred_element_type=jnp.float32)
    acc += jnp.dot(ea_ref[...], w1c_ref[...], preferred_element_type=jnp.float32)
    acc += jnp.dot(onehot, up_ref[...], preferred_element_type=jnp.float32)
    h = jnp.maximum(acc, 0.0)
    out_ref[...] = jnp.dot(h, w2_ref[...], preferred_element_type=jnp.float32) + b2_ref[...]


def kernel(src, dest, edge_attr, u, batch, W1, b1, W2, b2):
    e = src.shape[0]
    g = u.shape[0]
    nb = e // BE
    w1a = W1[:NODE_F]
    w1b = W1[NODE_F:2 * NODE_F]
    w1c = W1[2 * NODE_F:2 * NODE_F + EDGE_F]
    w1d = W1[2 * NODE_F + EDGE_F:]
    up = u @ w1d + b1[None, :]
    batch3 = batch.astype(jnp.int32).reshape(nb, 1, BE)
    b2r = b2.reshape(1, EDGE_F)

    grid_spec = pl.GridSpec(
        grid=(nb,),
        in_specs=[
            pl.BlockSpec((1, 1, BE), lambda i: (i, 0, 0)),
            pl.BlockSpec((BE, NODE_F), lambda i: (i, 0)),
            pl.BlockSpec((BE, NODE_F), lambda i: (i, 0)),
            pl.BlockSpec((BE, EDGE_F), lambda i: (i, 0)),
            pl.BlockSpec((g, HIDDEN), lambda i: (0, 0)),
            pl.BlockSpec((NODE_F, HIDDEN), lambda i: (0, 0)),
            pl.BlockSpec((NODE_F, HIDDEN), lambda i: (0, 0)),
            pl.BlockSpec((EDGE_F, HIDDEN), lambda i: (0, 0)),
            pl.BlockSpec((HIDDEN, EDGE_F), lambda i: (0, 0)),
            pl.BlockSpec((1, EDGE_F), lambda i: (0, 0)),
        ],
        out_specs=pl.BlockSpec((BE, EDGE_F), lambda i: (i, 0)),
    )
    return pl.pallas_call(
        _edge_mlp_kernel,
        grid_spec=grid_spec,
        out_shape=jax.ShapeDtypeStruct((e, EDGE_F), jnp.float32),
        compiler_params=pltpu.CompilerParams(
            dimension_semantics=("parallel",),
        ),
    )(batch3, src, dest, edge_attr, up, w1a, w1b, w1c, W2, b2r)
